# baseline (speedup 1.0000x reference)
import jax
import jax.numpy as jnp
from jax import lax
from jax.experimental import pallas as pl
from jax.experimental.pallas import tpu as pltpu
from jax.experimental.pallas import tpu_sc as plsc

VOCAB = 24
EMBED_DIM = 128
BATCH = 256
SEQ = 1024

NC = 2
NS = 16
NW = NC * NS

TOTAL = BATCH * SEQ
PER_W = TOTAL // NW
GROUP = 128
NGROUPS = PER_W // GROUP
NBUF = 4
LANES = 16


def _emb_kernel(table_hbm, idx_hbm, out_hbm, table_v, idx_v, obuf, osem):
    wid = lax.axis_index("s") * NC + lax.axis_index("c")
    pltpu.sync_copy(table_hbm, table_v)
    pltpu.sync_copy(idx_hbm.at[wid], idx_v)
    base = wid * PER_W * EMBED_DIM
    lane_off = jnp.arange(LANES, dtype=jnp.int32) * EMBED_DIM

    def wait_out():
        pltpu.make_async_copy(
            obuf.at[pl.ds(0, GROUP * EMBED_DIM)],
            out_hbm.at[pl.ds(base, GROUP * EMBED_DIM)],
            osem,
        ).wait()

    def body(g, _):
        b = lax.rem(g, NBUF)

        @pl.when(g >= NBUF)
        def _():
            wait_out()

        def block(i, _):
            j0 = i * LANES
            idxv = idx_v[g, pl.ds(j0, LANES)]
            gidx0 = idxv * EMBED_DIM
            sidx0 = (b * GROUP + j0) * EMBED_DIM + lane_off

            @plsc.parallel_loop(0, EMBED_DIM, step=1, unroll=8)
            def _(c):
                vals = plsc.load_gather(table_v, [gidx0 + c])
                plsc.store_scatter(obuf, [sidx0 + c], vals)

            return 0

        lax.fori_loop(0, GROUP // LANES, block, 0)
        pltpu.async_copy(
            obuf.at[pl.ds(b * GROUP * EMBED_DIM, GROUP * EMBED_DIM)],
            out_hbm.at[pl.ds(base + g * GROUP * EMBED_DIM, GROUP * EMBED_DIM)],
            osem,
        )
        return 0

    lax.fori_loop(0, NGROUPS, body, 0)

    for _ in range(min(NBUF, NGROUPS)):
        wait_out()


def kernel(x, table):
    idx = x.reshape(NW, NGROUPS, GROUP)
    mesh = plsc.VectorSubcoreMesh(core_axis_name="c", subcore_axis_name="s")
    out = pl.kernel(
        _emb_kernel,
        mesh=mesh,
        compiler_params=pltpu.CompilerParams(needs_layout_passes=False),
        out_type=jax.ShapeDtypeStruct((TOTAL * EMBED_DIM,), jnp.float32),
        scratch_types=[
            pltpu.VMEM((VOCAB * EMBED_DIM,), jnp.float32),
            pltpu.VMEM((NGROUPS, GROUP), jnp.int32),
            pltpu.VMEM((NBUF * GROUP * EMBED_DIM,), jnp.float32),
            pltpu.SemaphoreType.DMA,
        ],
    )(table.reshape(VOCAB * EMBED_DIM), idx)
    return out.reshape(BATCH, SEQ, EMBED_DIM)


# speedup vs baseline: 2.5486x; 2.5486x over previous
import jax
import jax.numpy as jnp
import numpy as np
from jax import lax
from jax.experimental import pallas as pl
from jax.experimental.pallas import tpu as pltpu
from jax.experimental.pallas import tpu_sc as plsc

VOCAB = 24
EMBED_DIM = 128
BATCH = 256
SEQ = 1024

NC = 2
NS = 16
NW = NC * NS

TOTAL = BATCH * SEQ
PER_W = TOTAL // NW
GROUP = 128
NGROUPS = PER_W // GROUP
NBUF = 4
LANES = 16


def _emb_kernel(table_hbm, idx_hbm, out_hbm, table_v, idx_v, obuf, osem):
    wid = lax.axis_index("s") * NC + lax.axis_index("c")
    pltpu.sync_copy(table_hbm, table_v)
    pltpu.sync_copy(idx_hbm.at[wid], idx_v)
    base = wid * PER_W * EMBED_DIM
    lane_off = jnp.arange(LANES, dtype=jnp.int32) * EMBED_DIM

    def wait_out():
        pltpu.make_async_copy(
            obuf.at[pl.ds(0, GROUP * EMBED_DIM)],
            out_hbm.at[pl.ds(base, GROUP * EMBED_DIM)],
            osem,
        ).wait()

    def body(g, _):
        b = lax.rem(g, NBUF)

        @pl.when(g >= NBUF)
        def _():
            wait_out()

        def block(i, _):
            j0 = i * LANES
            idxv = idx_v[g, pl.ds(j0, LANES)]
            gbase = idxv * EMBED_DIM
            sbase = (b * GROUP + j0) * EMBED_DIM + lane_off

            @plsc.parallel_loop(0, EMBED_DIM // LANES, step=1, unroll=2)
            def _(t):
                iota = lax.iota(jnp.int32, LANES)
                c0 = t * LANES
                gb = gbase + c0
                sb = sbase + c0
                for d in range(LANES):
                    ofs = (iota + d) & (LANES - 1)
                    vals = plsc.load_gather(table_v, [gb + ofs])
                    plsc.store_scatter(obuf, [sb + ofs], vals)

            return 0

        lax.fori_loop(0, GROUP // LANES, block, 0)
        pltpu.async_copy(
            obuf.at[pl.ds(b * GROUP * EMBED_DIM, GROUP * EMBED_DIM)],
            out_hbm.at[pl.ds(base + g * GROUP * EMBED_DIM, GROUP * EMBED_DIM)],
            osem,
        )
        return 0

    lax.fori_loop(0, NGROUPS, body, 0)

    for _ in range(min(NBUF, NGROUPS)):
        wait_out()


def kernel(x, table):
    idx = x.reshape(NW, NGROUPS, GROUP)
    mesh = plsc.VectorSubcoreMesh(core_axis_name="c", subcore_axis_name="s")
    out = pl.kernel(
        _emb_kernel,
        mesh=mesh,
        compiler_params=pltpu.CompilerParams(needs_layout_passes=False),
        out_type=jax.ShapeDtypeStruct((TOTAL * EMBED_DIM,), jnp.float32),
        scratch_types=[
            pltpu.VMEM((VOCAB * EMBED_DIM,), jnp.float32),
            pltpu.VMEM((NGROUPS, GROUP), jnp.int32),
            pltpu.VMEM((NBUF * GROUP * EMBED_DIM,), jnp.float32),
            pltpu.SemaphoreType.DMA,
        ],
    )(table.reshape(VOCAB * EMBED_DIM), idx)
    return out.reshape(BATCH, SEQ, EMBED_DIM)


# speedup vs baseline: 3.1375x; 1.2311x over previous
import jax
import jax.numpy as jnp
from jax import lax
from jax.experimental import pallas as pl
from jax.experimental.pallas import tpu as pltpu
from jax.experimental.pallas import tpu_sc as plsc

VOCAB = 24
EMBED_DIM = 128
BATCH = 256
SEQ = 1024

NC = 2
NS = 16
NW = NC * NS

TOTAL = BATCH * SEQ
PER_W = TOTAL // NW
GROUP = 128
NGROUPS = PER_W // GROUP
LANES = 16

REPS = 8
GDEPTH = 4
PERIOD = 16
G_PER = 9
C_PER = PERIOD - G_PER
NGATHER = NGROUPS // PERIOD * G_PER
NCOMP = NGROUPS - NGATHER
CBUF = 2
GROUP_ELEMS = GROUP * EMBED_DIM


def _emb_kernel(tab_rep_hbm, tab_hbm, idx_hbm, out_hbm,
                table_v, idx_v, rows_v, cbuf, gsem, g_osem, c_osem):
    wid = lax.axis_index("s") * NC + lax.axis_index("c")
    pltpu.sync_copy(tab_hbm, table_v)
    pltpu.sync_copy(idx_hbm.at[wid], idx_v)
    base = wid * PER_W

    def fire_gather(k, b):
        rep = wid * REPS + lax.rem(k, REPS)
        pltpu.async_copy(tab_rep_hbm.at[rep].at[idx_v.at[k]], rows_v.at[b], gsem)

    def wait_gather(b):
        pltpu.make_async_copy(
            tab_rep_hbm.at[0].at[idx_v.at[0]], rows_v.at[b], gsem
        ).wait()

    def fire_gout(k, b):
        pltpu.async_copy(
            rows_v.at[b],
            out_hbm.at[pl.ds(base + k * GROUP, GROUP)],
            g_osem,
        )

    def wait_gout():
        pltpu.make_async_copy(
            rows_v.at[0], out_hbm.at[pl.ds(base, GROUP)], g_osem
        ).wait()

    def wait_cout():
        pltpu.make_async_copy(
            cbuf.at[pl.ds(0, GROUP)],
            out_hbm.at[pl.ds(base, GROUP)],
            c_osem,
        ).wait()

    def do_compute(c):
        g = NGATHER + c
        cb = lax.rem(c, CBUF)

        def block(i, _):
            j0 = i * LANES
            idxv = idx_v[g, pl.ds(j0, LANES)]
            rowv = cb * GROUP + j0 + lax.iota(jnp.int32, LANES)

            @plsc.parallel_loop(0, EMBED_DIM // LANES, step=1, unroll=2)
            def _(t):
                iota = lax.iota(jnp.int32, LANES)
                c0 = t * LANES
                for d in range(LANES):
                    colv = ((iota + d) & (LANES - 1)) + c0
                    vals = plsc.load_gather(table_v, [idxv, colv])
                    plsc.store_scatter(cbuf, [rowv, colv], vals)

            return 0

        lax.fori_loop(0, GROUP // LANES, block, 0)
        pltpu.async_copy(
            cbuf.at[pl.ds(cb * GROUP, GROUP)],
            out_hbm.at[pl.ds(base + g * GROUP, GROUP)],
            c_osem,
        )

    fire_gather(0, 0)
    fire_gather(1, 1)

    def body(m, _):
        period, slot = lax.div(m, PERIOD), lax.rem(m, PERIOD)
        is_gather = slot < G_PER

        @pl.when(is_gather)
        def _():
            k = period * G_PER + slot
            b = lax.rem(k, GDEPTH)
            wait_gather(b)
            fire_gout(k, b)

            @pl.when(k >= 2)
            def _():
                wait_gout()

            @pl.when(k + 2 < NGATHER)
            def _():
                fire_gather(k + 2, lax.rem(k + 2, GDEPTH))

        @pl.when(jnp.logical_not(is_gather))
        def _():
            c = period * C_PER + (slot - G_PER)

            @pl.when(c >= CBUF)
            def _():
                wait_cout()

            do_compute(c)

        return 0

    lax.fori_loop(0, NGROUPS, body, 0)

    for _ in range(2):
        wait_gout()
    for _ in range(CBUF):
        wait_cout()


def kernel(x, table):
    idx = x.reshape(NW, NGROUPS, GROUP)
    table_rep = jnp.tile(table[None], (NW * REPS, 1, 1))
    mesh = plsc.VectorSubcoreMesh(core_axis_name="c", subcore_axis_name="s")
    out = pl.kernel(
        _emb_kernel,
        mesh=mesh,
        compiler_params=pltpu.CompilerParams(needs_layout_passes=False),
        out_type=jax.ShapeDtypeStruct((TOTAL, EMBED_DIM), jnp.float32),
        scratch_types=[
            pltpu.VMEM((VOCAB, EMBED_DIM), jnp.float32),
            pltpu.VMEM((NGROUPS, GROUP), jnp.int32),
            pltpu.VMEM((GDEPTH, GROUP, EMBED_DIM), jnp.float32),
            pltpu.VMEM((CBUF * GROUP, EMBED_DIM), jnp.float32),
            pltpu.SemaphoreType.DMA,
            pltpu.SemaphoreType.DMA,
            pltpu.SemaphoreType.DMA,
        ],
    )(table_rep, table, idx)
    return out.reshape(BATCH, SEQ, EMBED_DIM)


# speedup vs baseline: 4.3512x; 1.3868x over previous
import jax
import jax.numpy as jnp
from jax import lax
from jax.experimental import pallas as pl
from jax.experimental.pallas import tpu as pltpu
from jax.experimental.pallas import tpu_sc as plsc

VOCAB = 24
EMBED_DIM = 128
BATCH = 256
SEQ = 1024

NC = 2
NS = 16
NW = NC * NS

TOTAL = BATCH * SEQ
PER_W = TOTAL // NW
GROUP = 128
NGROUPS = PER_W // GROUP
LANES = 16

REPS = 8
GDEPTH = 5
PERIOD = 7
NGATHER = 37
NCOMP = NGROUPS - NGATHER
CBUF = 2
GROUP_ELEMS = GROUP * EMBED_DIM


def _emb_kernel(tab_rep_hbm, tab_hbm, idx_hbm, out_hbm,
                table_v, idx_v, rows_v, cbuf, gsem, g_osem, c_osem):
    wid = lax.axis_index("s") * NC + lax.axis_index("c")
    pltpu.sync_copy(tab_hbm, table_v)
    pltpu.sync_copy(idx_hbm.at[wid], idx_v)
    base = wid * PER_W

    def fire_gather(k, b):
        rep = wid * REPS + lax.rem(k, REPS)
        pltpu.async_copy(tab_rep_hbm.at[rep].at[idx_v.at[k]], rows_v.at[b], gsem)

    def wait_gather(b):
        pltpu.make_async_copy(
            tab_rep_hbm.at[0].at[idx_v.at[0]], rows_v.at[b], gsem
        ).wait()

    def fire_gout(k, b):
        pltpu.async_copy(
            rows_v.at[b],
            out_hbm.at[pl.ds(base + k * GROUP, GROUP)],
            g_osem,
        )

    def wait_gout():
        pltpu.make_async_copy(
            rows_v.at[0], out_hbm.at[pl.ds(base, GROUP)], g_osem
        ).wait()

    def wait_cout():
        pltpu.make_async_copy(
            cbuf.at[pl.ds(0, GROUP)],
            out_hbm.at[pl.ds(base, GROUP)],
            c_osem,
        ).wait()

    def do_compute(c):
        g = NGATHER + c
        cb = lax.rem(c, CBUF)

        def block(i, _):
            j0 = i * LANES
            idxv = idx_v[g, pl.ds(j0, LANES)]
            rowv = cb * GROUP + j0 + lax.iota(jnp.int32, LANES)

            @plsc.parallel_loop(0, EMBED_DIM // LANES, step=1, unroll=2)
            def _(t):
                iota = lax.iota(jnp.int32, LANES)
                c0 = t * LANES
                for d in range(LANES):
                    colv = ((iota + d) & (LANES - 1)) + c0
                    vals = plsc.load_gather(table_v, [idxv, colv])
                    plsc.store_scatter(cbuf, [rowv, colv], vals)

            return 0

        lax.fori_loop(0, GROUP // LANES, block, 0)
        pltpu.async_copy(
            cbuf.at[pl.ds(cb * GROUP, GROUP)],
            out_hbm.at[pl.ds(base + g * GROUP, GROUP)],
            c_osem,
        )

    fire_gather(0, 0)
    fire_gather(1, 1)
    fire_gather(2, 2)

    def body(m, _):
        period, slot = lax.div(m, PERIOD), lax.rem(m, PERIOD)
        is_gather = lax.rem(slot, 2) == 0

        @pl.when(is_gather)
        def _():
            k = period * 4 + lax.div(slot, 2)
            b = lax.rem(k, GDEPTH)
            wait_gather(b)
            fire_gout(k, b)

            @pl.when(k >= 2)
            def _():
                wait_gout()

            @pl.when(k + 3 < NGATHER)
            def _():
                fire_gather(k + 3, lax.rem(k + 3, GDEPTH))

        @pl.when(jnp.logical_not(is_gather))
        def _():
            c = period * 3 + lax.div(slot, 2)

            @pl.when(c >= CBUF)
            def _():
                wait_cout()

            do_compute(c)

        return 0

    lax.fori_loop(0, NGROUPS, body, 0)

    for _ in range(2):
        wait_gout()
    for _ in range(CBUF):
        wait_cout()


def kernel(x, table):
    idx = x.reshape(NW, NGROUPS, GROUP)
    table_rep = jnp.tile(table[None], (NW * REPS, 1, 1))
    mesh = plsc.VectorSubcoreMesh(core_axis_name="c", subcore_axis_name="s")
    out = pl.kernel(
        _emb_kernel,
        mesh=mesh,
        compiler_params=pltpu.CompilerParams(needs_layout_passes=False),
        out_type=jax.ShapeDtypeStruct((TOTAL, EMBED_DIM), jnp.float32),
        scratch_types=[
            pltpu.VMEM((VOCAB, EMBED_DIM), jnp.float32),
            pltpu.VMEM((NGROUPS, GROUP), jnp.int32),
            pltpu.VMEM((GDEPTH, GROUP, EMBED_DIM), jnp.float32),
            pltpu.VMEM((CBUF * GROUP, EMBED_DIM), jnp.float32),
            pltpu.SemaphoreType.DMA,
            pltpu.SemaphoreType.DMA,
            pltpu.SemaphoreType.DMA,
        ],
    )(table_rep, table, idx)
    return out.reshape(BATCH, SEQ, EMBED_DIM)


# speedup vs baseline: 4.5379x; 1.0429x over previous
import jax
import jax.numpy as jnp
from jax import lax
from jax.experimental import pallas as pl
from jax.experimental.pallas import tpu as pltpu
from jax.experimental.pallas import tpu_sc as plsc

VOCAB = 24
EMBED_DIM = 128
BATCH = 256
SEQ = 1024

NC = 2
NS = 16
NW = NC * NS

TOTAL = BATCH * SEQ
PER_W = TOTAL // NW
GROUP = 128
NGROUPS = PER_W // GROUP
LANES = 16

REPS = 8
GDEPTH = 5
PERIOD = 7
NGATHER = 37
NCOMP = NGROUPS - NGATHER
CBUF = 2
GROUP_ELEMS = GROUP * EMBED_DIM


def _emb_kernel(tab_rep_hbm, tab_hbm, idx_hbm, out_hbm,
                table_v, idx_v, rows_v, cbuf, gsem, g_osem, c_osem):
    wid = lax.axis_index("s") * NC + lax.axis_index("c")
    pltpu.sync_copy(tab_hbm, table_v)
    pltpu.sync_copy(idx_hbm.at[wid], idx_v)
    base = wid * PER_W

    def fire_gather(k, b):
        rep = wid * REPS + lax.rem(k, REPS)
        pltpu.async_copy(tab_rep_hbm.at[rep].at[idx_v.at[k]], rows_v.at[b], gsem)

    def wait_gather(b):
        pltpu.make_async_copy(
            tab_rep_hbm.at[0].at[idx_v.at[0]], rows_v.at[b], gsem
        ).wait()

    def fire_gout(k, b):
        pltpu.async_copy(
            rows_v.at[b],
            out_hbm.at[pl.ds(base + k * GROUP, GROUP)],
            g_osem,
        )

    def wait_gout():
        pltpu.make_async_copy(
            rows_v.at[0], out_hbm.at[pl.ds(base, GROUP)], g_osem
        ).wait()

    def wait_cout():
        pltpu.make_async_copy(
            cbuf.at[pl.ds(0, GROUP)],
            out_hbm.at[pl.ds(base, GROUP)],
            c_osem,
        ).wait()

    def do_compute(c):
        g = NGATHER + c
        cb = lax.rem(c, CBUF)

        def block(i, _):
            j0 = i * LANES
            idxv = idx_v[g, pl.ds(j0, LANES)]
            rowv = cb * GROUP + j0 + lax.iota(jnp.int32, LANES)

            @plsc.parallel_loop(0, EMBED_DIM, step=1, unroll=8)
            def _(u):
                iota = lax.iota(jnp.int32, LANES)
                colv = ((iota + u) & (LANES - 1)) + (u & ~(LANES - 1))
                vals = plsc.load_gather(table_v, [idxv, colv])
                plsc.store_scatter(cbuf, [rowv, colv], vals)

            return 0

        lax.fori_loop(0, GROUP // LANES, block, 0)
        pltpu.async_copy(
            cbuf.at[pl.ds(cb * GROUP, GROUP)],
            out_hbm.at[pl.ds(base + g * GROUP, GROUP)],
            c_osem,
        )

    fire_gather(0, 0)
    fire_gather(1, 1)
    fire_gather(2, 2)

    def body(m, _):
        period, slot = lax.div(m, PERIOD), lax.rem(m, PERIOD)
        is_gather = lax.rem(slot, 2) == 0

        @pl.when(is_gather)
        def _():
            k = period * 4 + lax.div(slot, 2)
            b = lax.rem(k, GDEPTH)
            wait_gather(b)
            fire_gout(k, b)

            @pl.when(k >= 2)
            def _():
                wait_gout()

            @pl.when(k + 3 < NGATHER)
            def _():
                fire_gather(k + 3, lax.rem(k + 3, GDEPTH))

        @pl.when(jnp.logical_not(is_gather))
        def _():
            c = period * 3 + lax.div(slot, 2)

            @pl.when(c >= CBUF)
            def _():
                wait_cout()

            do_compute(c)

        return 0

    lax.fori_loop(0, NGROUPS, body, 0)

    for _ in range(2):
        wait_gout()
    for _ in range(CBUF):
        wait_cout()


def kernel(x, table):
    idx = x.reshape(NW, NGROUPS, GROUP)
    table_rep = jnp.tile(table[None], (NW * REPS, 1, 1))
    mesh = plsc.VectorSubcoreMesh(core_axis_name="c", subcore_axis_name="s")
    out = pl.kernel(
        _emb_kernel,
        mesh=mesh,
        compiler_params=pltpu.CompilerParams(needs_layout_passes=False),
        out_type=jax.ShapeDtypeStruct((TOTAL, EMBED_DIM), jnp.float32),
        scratch_types=[
            pltpu.VMEM((VOCAB, EMBED_DIM), jnp.float32),
            pltpu.VMEM((NGROUPS, GROUP), jnp.int32),
            pltpu.VMEM((GDEPTH, GROUP, EMBED_DIM), jnp.float32),
            pltpu.VMEM((CBUF * GROUP, EMBED_DIM), jnp.float32),
            pltpu.SemaphoreType.DMA,
            pltpu.SemaphoreType.DMA,
            pltpu.SemaphoreType.DMA,
        ],
    )(table_rep, table, idx)
    return out.reshape(BATCH, SEQ, EMBED_DIM)


# speedup vs baseline: 4.9227x; 1.0848x over previous
import jax
import jax.numpy as jnp
from jax import lax
from jax.experimental import pallas as pl
from jax.experimental.pallas import tpu as pltpu
from jax.experimental.pallas import tpu_sc as plsc

VOCAB = 24
EMBED_DIM = 128
BATCH = 256
SEQ = 1024

NC = 2
NS = 16
NW = NC * NS

TOTAL = BATCH * SEQ
PER_W = TOTAL // NW
GROUP = 128
NGROUPS = PER_W // GROUP
LANES = 16

REPS = 8
GDEPTH = 5
PERIOD = 7
NGATHER = 27
NCOMP = NGROUPS - NGATHER
CBUF = 2
GROUP_ELEMS = GROUP * EMBED_DIM


def _emb_kernel(tab_rep_hbm, tab_hbm, idx_hbm, out_hbm,
                table_v, idx_v, rows_v, cbuf, gsem, g_osem, c_osem):
    wid = lax.axis_index("s") * NC + lax.axis_index("c")
    pltpu.sync_copy(tab_hbm, table_v)
    pltpu.sync_copy(idx_hbm.at[wid], idx_v)
    base = wid * PER_W

    def fire_gather(k, b):
        rep = wid * REPS + lax.rem(k, REPS)
        pltpu.async_copy(tab_rep_hbm.at[rep].at[idx_v.at[k]], rows_v.at[b], gsem)

    def wait_gather(b):
        pltpu.make_async_copy(
            tab_rep_hbm.at[0].at[idx_v.at[0]], rows_v.at[b], gsem
        ).wait()

    def fire_gout(k, b):
        pltpu.async_copy(
            rows_v.at[b],
            out_hbm.at[pl.ds(base + k * GROUP, GROUP)],
            g_osem,
        )

    def wait_gout():
        pltpu.make_async_copy(
            rows_v.at[0], out_hbm.at[pl.ds(base, GROUP)], g_osem
        ).wait()

    def wait_cout():
        pltpu.make_async_copy(
            cbuf.at[pl.ds(0, GROUP)],
            out_hbm.at[pl.ds(base, GROUP)],
            c_osem,
        ).wait()

    def do_compute(c):
        g = NGATHER + c
        cb = lax.rem(c, CBUF)

        def block(i, _):
            j0 = i * LANES
            idxv = idx_v[g, pl.ds(j0, LANES)]
            rowv = cb * GROUP + j0 + lax.iota(jnp.int32, LANES)

            @plsc.parallel_loop(0, EMBED_DIM, step=1, unroll=8)
            def _(u):
                iota = lax.iota(jnp.int32, LANES)
                colv = ((iota + u) & (LANES - 1)) + (u & ~(LANES - 1))
                vals = plsc.load_gather(table_v, [idxv, colv])
                plsc.store_scatter(cbuf, [rowv, colv], vals)

            return 0

        lax.fori_loop(0, GROUP // LANES, block, 0)
        pltpu.async_copy(
            cbuf.at[pl.ds(cb * GROUP, GROUP)],
            out_hbm.at[pl.ds(base + g * GROUP, GROUP)],
            c_osem,
        )

    fire_gather(0, 0)
    fire_gather(1, 1)
    fire_gather(2, 2)

    def body(m, _):
        period, slot = lax.div(m, PERIOD), lax.rem(m, PERIOD)
        is_gather = lax.rem(slot, 2) == 1

        @pl.when(is_gather)
        def _():
            k = period * 3 + lax.div(slot, 2)
            b = lax.rem(k, GDEPTH)
            wait_gather(b)
            fire_gout(k, b)

            @pl.when(k >= 2)
            def _():
                wait_gout()

            @pl.when(k + 3 < NGATHER)
            def _():
                fire_gather(k + 3, lax.rem(k + 3, GDEPTH))

        @pl.when(jnp.logical_not(is_gather))
        def _():
            c = period * 4 + lax.div(slot, 2)

            @pl.when(c >= CBUF)
            def _():
                wait_cout()

            do_compute(c)

        return 0

    lax.fori_loop(0, NGROUPS, body, 0)

    for _ in range(2):
        wait_gout()
    for _ in range(CBUF):
        wait_cout()


def kernel(x, table):
    idx = x.reshape(NW, NGROUPS, GROUP)
    table_rep = jnp.tile(table[None], (NW * REPS, 1, 1))
    mesh = plsc.VectorSubcoreMesh(core_axis_name="c", subcore_axis_name="s")
    out = pl.kernel(
        _emb_kernel,
        mesh=mesh,
        compiler_params=pltpu.CompilerParams(needs_layout_passes=False),
        out_type=jax.ShapeDtypeStruct((TOTAL, EMBED_DIM), jnp.float32),
        scratch_types=[
            pltpu.VMEM((VOCAB, EMBED_DIM), jnp.float32),
            pltpu.VMEM((NGROUPS, GROUP), jnp.int32),
            pltpu.VMEM((GDEPTH, GROUP, EMBED_DIM), jnp.float32),
            pltpu.VMEM((CBUF * GROUP, EMBED_DIM), jnp.float32),
            pltpu.SemaphoreType.DMA,
            pltpu.SemaphoreType.DMA,
            pltpu.SemaphoreType.DMA,
        ],
    )(table_rep, table, idx)
    return out.reshape(BATCH, SEQ, EMBED_DIM)


# speedup vs baseline: 5.1665x; 1.0495x over previous
import jax
import jax.numpy as jnp
from jax import lax
from jax.experimental import pallas as pl
from jax.experimental.pallas import tpu as pltpu
from jax.experimental.pallas import tpu_sc as plsc

VOCAB = 24
EMBED_DIM = 128
BATCH = 256
SEQ = 1024

NC = 2
NS = 16
NW = NC * NS

TOTAL = BATCH * SEQ
PER_W = TOTAL // NW
GROUP = 128
NGROUPS = PER_W // GROUP
LANES = 16

REPS = 8
GDEPTH = 5
PERIOD = 3
NGATHER = 21
NCOMP = NGROUPS - NGATHER
CBUF = 2
GROUP_ELEMS = GROUP * EMBED_DIM


def _emb_kernel(tab_rep_hbm, tab_hbm, idx_hbm, out_hbm,
                table_v, idx_v, rows_v, cbuf, gsem, g_osem, c_osem):
    wid = lax.axis_index("s") * NC + lax.axis_index("c")
    pltpu.sync_copy(tab_hbm, table_v)
    pltpu.sync_copy(idx_hbm.at[wid], idx_v)
    base = wid * PER_W

    def fire_gather(k, b):
        rep = wid * REPS + lax.rem(k, REPS)
        pltpu.async_copy(tab_rep_hbm.at[rep].at[idx_v.at[k]], rows_v.at[b], gsem)

    def wait_gather(b):
        pltpu.make_async_copy(
            tab_rep_hbm.at[0].at[idx_v.at[0]], rows_v.at[b], gsem
        ).wait()

    def fire_gout(k, b):
        pltpu.async_copy(
            rows_v.at[b],
            out_hbm.at[pl.ds(base + k * GROUP, GROUP)],
            g_osem,
        )

    def wait_gout():
        pltpu.make_async_copy(
            rows_v.at[0], out_hbm.at[pl.ds(base, GROUP)], g_osem
        ).wait()

    def wait_cout():
        pltpu.make_async_copy(
            cbuf.at[pl.ds(0, GROUP)],
            out_hbm.at[pl.ds(base, GROUP)],
            c_osem,
        ).wait()

    def do_compute(c):
        g = NGATHER + c
        cb = lax.rem(c, CBUF)

        def block(i, _):
            j0 = i * LANES
            idxv = idx_v[g, pl.ds(j0, LANES)]
            rowv = cb * GROUP + j0 + lax.iota(jnp.int32, LANES)

            @plsc.parallel_loop(0, EMBED_DIM, step=1, unroll=8)
            def _(u):
                iota = lax.iota(jnp.int32, LANES)
                colv = ((iota + u) & (LANES - 1)) + (u & ~(LANES - 1))
                vals = plsc.load_gather(table_v, [idxv, colv])
                plsc.store_scatter(cbuf, [rowv, colv], vals)

            return 0

        lax.fori_loop(0, GROUP // LANES, block, 0)
        pltpu.async_copy(
            cbuf.at[pl.ds(cb * GROUP, GROUP)],
            out_hbm.at[pl.ds(base + g * GROUP, GROUP)],
            c_osem,
        )

    fire_gather(0, 0)
    fire_gather(1, 1)
    fire_gather(2, 2)

    def body(m, _):
        period, slot = lax.div(m, PERIOD), lax.rem(m, PERIOD)
        is_gather = lax.rem(slot, 2) == 1

        @pl.when(is_gather)
        def _():
            k = period
            b = lax.rem(k, GDEPTH)
            wait_gather(b)
            fire_gout(k, b)

            @pl.when(k >= 2)
            def _():
                wait_gout()

            @pl.when(k + 3 < NGATHER)
            def _():
                fire_gather(k + 3, lax.rem(k + 3, GDEPTH))

        @pl.when(jnp.logical_not(is_gather))
        def _():
            c = period * 2 + lax.div(slot, 2)

            @pl.when(c >= CBUF)
            def _():
                wait_cout()

            do_compute(c)

        return 0

    lax.fori_loop(0, NGROUPS, body, 0)

    for _ in range(2):
        wait_gout()
    for _ in range(CBUF):
        wait_cout()


def kernel(x, table):
    idx = x.reshape(NW, NGROUPS, GROUP)
    table_rep = jnp.tile(table[None], (NW * REPS, 1, 1))
    mesh = plsc.VectorSubcoreMesh(core_axis_name="c", subcore_axis_name="s")
    out = pl.kernel(
        _emb_kernel,
        mesh=mesh,
        compiler_params=pltpu.CompilerParams(needs_layout_passes=False),
        out_type=jax.ShapeDtypeStruct((TOTAL, EMBED_DIM), jnp.float32),
        scratch_types=[
            pltpu.VMEM((VOCAB, EMBED_DIM), jnp.float32),
            pltpu.VMEM((NGROUPS, GROUP), jnp.int32),
            pltpu.VMEM((GDEPTH, GROUP, EMBED_DIM), jnp.float32),
            pltpu.VMEM((CBUF * GROUP, EMBED_DIM), jnp.float32),
            pltpu.SemaphoreType.DMA,
            pltpu.SemaphoreType.DMA,
            pltpu.SemaphoreType.DMA,
        ],
    )(table_rep, table, idx)
    return out.reshape(BATCH, SEQ, EMBED_DIM)


# speedup vs baseline: 5.2975x; 1.0254x over previous
import jax
import jax.numpy as jnp
from jax import lax
from jax.experimental import pallas as pl
from jax.experimental.pallas import tpu as pltpu
from jax.experimental.pallas import tpu_sc as plsc

VOCAB = 24
EMBED_DIM = 128
BATCH = 256
SEQ = 1024

NC = 2
NS = 16
NW = NC * NS

TOTAL = BATCH * SEQ
PER_W = TOTAL // NW
GROUP = 128
NGROUPS = PER_W // GROUP
LANES = 16

REPS = 8
GDEPTH = 5
PERIOD = 4
NGATHER = 16
NCOMP = NGROUPS - NGATHER
CBUF = 2
GROUP_ELEMS = GROUP * EMBED_DIM


def _emb_kernel(tab_rep_hbm, tab_hbm, idx_hbm, out_hbm,
                table_v, idx_v, rows_v, cbuf, gsem, g_osem, c_osem):
    wid = lax.axis_index("s") * NC + lax.axis_index("c")
    pltpu.sync_copy(tab_hbm, table_v)
    pltpu.sync_copy(idx_hbm.at[wid], idx_v)
    base = wid * PER_W

    def fire_gather(k, b):
        rep = wid * REPS + lax.rem(k, REPS)
        pltpu.async_copy(tab_rep_hbm.at[rep].at[idx_v.at[k]], rows_v.at[b], gsem)

    def wait_gather(b):
        pltpu.make_async_copy(
            tab_rep_hbm.at[0].at[idx_v.at[0]], rows_v.at[b], gsem
        ).wait()

    def fire_gout(k, b):
        pltpu.async_copy(
            rows_v.at[b],
            out_hbm.at[pl.ds(base + k * GROUP, GROUP)],
            g_osem,
        )

    def wait_gout():
        pltpu.make_async_copy(
            rows_v.at[0], out_hbm.at[pl.ds(base, GROUP)], g_osem
        ).wait()

    def wait_cout():
        pltpu.make_async_copy(
            cbuf.at[pl.ds(0, GROUP)],
            out_hbm.at[pl.ds(base, GROUP)],
            c_osem,
        ).wait()

    def do_compute(c):
        g = NGATHER + c
        cb = lax.rem(c, CBUF)

        def block(i, _):
            j0 = i * LANES
            idxv = idx_v[g, pl.ds(j0, LANES)]
            rowv = cb * GROUP + j0 + lax.iota(jnp.int32, LANES)

            @plsc.parallel_loop(0, EMBED_DIM, step=1, unroll=8)
            def _(u):
                iota = lax.iota(jnp.int32, LANES)
                colv = ((iota + u) & (LANES - 1)) + (u & ~(LANES - 1))
                vals = plsc.load_gather(table_v, [idxv, colv])
                plsc.store_scatter(cbuf, [rowv, colv], vals)

            return 0

        lax.fori_loop(0, GROUP // LANES, block, 0)
        pltpu.async_copy(
            cbuf.at[pl.ds(cb * GROUP, GROUP)],
            out_hbm.at[pl.ds(base + g * GROUP, GROUP)],
            c_osem,
        )

    fire_gather(0, 0)
    fire_gather(1, 1)
    fire_gather(2, 2)

    def body(m, _):
        period, slot = lax.div(m, PERIOD), lax.rem(m, PERIOD)
        is_gather = slot == 1

        @pl.when(is_gather)
        def _():
            k = period
            b = lax.rem(k, GDEPTH)
            wait_gather(b)
            fire_gout(k, b)

            @pl.when(k >= 2)
            def _():
                wait_gout()

            @pl.when(k + 3 < NGATHER)
            def _():
                fire_gather(k + 3, lax.rem(k + 3, GDEPTH))

        @pl.when(jnp.logical_not(is_gather))
        def _():
            c = period * 3 + slot - jnp.where(slot > 1, 1, 0)

            @pl.when(c >= CBUF)
            def _():
                wait_cout()

            do_compute(c)

        return 0

    lax.fori_loop(0, NGROUPS, body, 0)

    for _ in range(2):
        wait_gout()
    for _ in range(CBUF):
        wait_cout()


def kernel(x, table):
    idx = x.reshape(NW, NGROUPS, GROUP)
    table_rep = jnp.tile(table[None], (NW * REPS, 1, 1))
    mesh = plsc.VectorSubcoreMesh(core_axis_name="c", subcore_axis_name="s")
    out = pl.kernel(
        _emb_kernel,
        mesh=mesh,
        compiler_params=pltpu.CompilerParams(needs_layout_passes=False),
        out_type=jax.ShapeDtypeStruct((TOTAL, EMBED_DIM), jnp.float32),
        scratch_types=[
            pltpu.VMEM((VOCAB, EMBED_DIM), jnp.float32),
            pltpu.VMEM((NGROUPS, GROUP), jnp.int32),
            pltpu.VMEM((GDEPTH, GROUP, EMBED_DIM), jnp.float32),
            pltpu.VMEM((CBUF * GROUP, EMBED_DIM), jnp.float32),
            pltpu.SemaphoreType.DMA,
            pltpu.SemaphoreType.DMA,
            pltpu.SemaphoreType.DMA,
        ],
    )(table_rep, table, idx)
    return out.reshape(BATCH, SEQ, EMBED_DIM)
